# Initial kernel scaffold; baseline (speedup 1.0000x reference)
import functools

import numpy as np
import jax
import jax.numpy as jnp
from jax import lax
from jax.experimental import pallas as pl
from jax.experimental.pallas import tpu as pltpu
from jax.experimental.pallas import tpu_sc as plsc

N_TOTAL = 100000
D_MODEL = 2048
D_EMB = 128
BATCH = 1024
M_NOISE = 4096
GAMMA = 0.5
TAU = 0.07
Z = np.float32(2876934.2 / 1281167 * N_TOTAL)
C0E = np.float32(M_NOISE * (1.0 / N_TOTAL) + 1e-07)
LOG_C0 = np.float32(np.log(M_NOISE * (1.0 / N_TOTAL)))

NC, NS = 2, 16
NW = NC * NS
NPAIR = BATCH * M_NOISE
NP_W = NPAIR // NW
CS = 8192
NCH = NP_W // CS
ROWS_W = BATCH // NW

_RIDX = np.asarray(
    jax.random.randint(jax.random.key(12345), (BATCH, M_NOISE), 0, N_TOTAL,
                       dtype=jnp.int32))
_GIDX_FLAT = (
    _RIDX.astype(np.int64) * BATCH
    + np.arange(BATCH, dtype=np.int64)[:, None]
).astype(np.int32).reshape(-1)


def _emb_body(o_ref, w_ref, b_ref, emb_ref, embh_ref):
    x = lax.dot_general(o_ref[...], w_ref[...], (((1,), (1,)), ((), ())),
                        preferred_element_type=jnp.float32,
                        precision=lax.Precision.HIGHEST)
    x = x + b_ref[...]
    e = x / jnp.sqrt(jnp.sum(x * x, axis=1, keepdims=True))
    emb_ref[...] = e
    embh_ref[...] = e.astype(jnp.bfloat16)


def _emb_kernel(outputs, W, b2):
    return pl.pallas_call(
        _emb_body,
        out_shape=(jax.ShapeDtypeStruct((BATCH, D_EMB), jnp.float32),
                   jax.ShapeDtypeStruct((BATCH, D_EMB), jnp.bfloat16)),
    )(outputs, W, b2)


_KB = 2000


def _score_body(mb_ref, embh_ref, s_ref):
    s_ref[...] = lax.dot_general(
        mb_ref[...].astype(jnp.bfloat16), embh_ref[...],
        (((1,), (1,)), ((), ())), preferred_element_type=jnp.float32)


def _score_kernel(memory_bank, embh):
    return pl.pallas_call(
        _score_body,
        grid=(N_TOTAL // _KB,),
        in_specs=[pl.BlockSpec((_KB, D_EMB), lambda i: (i, 0)),
                  pl.BlockSpec((BATCH, D_EMB), lambda i: (0, 0))],
        out_specs=pl.BlockSpec((_KB, BATCH), lambda i: (i, 0)),
        out_shape=jax.ShapeDtypeStruct((N_TOTAL, BATCH), jnp.float32),
    )(memory_bank, embh)


_sc_mesh = plsc.VectorSubcoreMesh(core_axis_name="c", subcore_axis_name="s")


@functools.partial(
    pl.kernel,
    mesh=_sc_mesh,
    out_type=jax.ShapeDtypeStruct((NPAIR,), jnp.float32),
    scratch_types=[
        pltpu.VMEM((CS,), jnp.int32),
        pltpu.VMEM((CS,), jnp.float32),
        pltpu.SemaphoreType.DMA,
    ],
)
def _noise_gather(s_hbm, gidx_hbm, out_hbm, idx_v, val_v, sem):
    wid = lax.axis_index("s") * NC + lax.axis_index("c")

    def chunk(ci, carry):
        base = wid * NP_W + ci * CS
        pltpu.sync_copy(gidx_hbm.at[pl.ds(base, CS)], idx_v)
        pltpu.async_copy(s_hbm.at[idx_v], val_v, sem).wait()
        pltpu.sync_copy(val_v, out_hbm.at[pl.ds(base, CS)])
        return carry

    lax.fori_loop(0, NCH, chunk, 0)


@functools.partial(
    pl.kernel,
    mesh=_sc_mesh,
    out_type=jax.ShapeDtypeStruct((BATCH, D_EMB), jnp.float32),
    scratch_types=[
        pltpu.VMEM((ROWS_W,), jnp.int32),
        pltpu.VMEM((ROWS_W, D_EMB), jnp.float32),
        pltpu.SemaphoreType.DMA,
    ],
)
def _row_gather(mb_hbm, idx_hbm, out_hbm, idx_v, rows_v, sem):
    wid = lax.axis_index("s") * NC + lax.axis_index("c")
    base = wid * ROWS_W
    pltpu.sync_copy(idx_hbm.at[pl.ds(base, ROWS_W)], idx_v)
    pltpu.async_copy(mb_hbm.at[idx_v], rows_v, sem).wait()
    pltpu.sync_copy(rows_v, out_hbm.at[pl.ds(base, ROWS_W)])


def _final_body(emb_ref, md_ref, ns_ref, entries_ref, sums_ref):
    e = emb_ref[...]
    md = md_ref[...]
    data_ip = jnp.sum(e * md, axis=1)
    dp = jnp.exp(data_ip / TAU) / Z
    sum_ld = jnp.sum(jnp.log(dp) - jnp.log(dp + C0E))
    ns = ns_ref[...]
    npr = jnp.exp(ns / TAU) / Z
    sum_ln = jnp.sum(LOG_C0 - jnp.log(npr + C0E))
    upd = GAMMA * md + (1.0 - GAMMA) * e
    entries_ref[...] = upd / jnp.sqrt(jnp.sum(upd * upd, axis=1,
                                              keepdims=True))
    row = lax.broadcasted_iota(jnp.int32, (8, 128), 0)
    sums_ref[...] = jnp.where(row == 0, sum_ld, sum_ln)


def _final_kernel(emb, mem_data, noise_s):
    return pl.pallas_call(
        _final_body,
        out_shape=(jax.ShapeDtypeStruct((BATCH, D_EMB), jnp.float32),
                   jax.ShapeDtypeStruct((8, 128), jnp.float32)),
    )(emb, mem_data, noise_s)


def kernel(outputs, indices, memory_bank, W, b):
    emb, embh = _emb_kernel(outputs, W, b.reshape(1, D_EMB))
    s = _score_kernel(memory_bank, embh)
    noise_flat = _noise_gather(s.reshape(-1), jnp.asarray(_GIDX_FLAT))
    mem_data = _row_gather(memory_bank, indices)
    entries, sums = _final_kernel(emb, mem_data,
                                  noise_flat.reshape(BATCH, M_NOISE))
    sum_ld = sums[0, 0]
    sum_ln = sums[1, 0]
    loss = jnp.reshape(-(sum_ld + sum_ln) / BATCH, (1,))
    data_loss = jnp.reshape(-sum_ld / BATCH, (1,))
    noise_loss = jnp.reshape(-sum_ln / BATCH, (1,))
    return (loss, entries, data_loss, noise_loss)


# speedup vs baseline: 25.3698x; 25.3698x over previous
import functools

import numpy as np
import jax
import jax.numpy as jnp
from jax import lax
from jax.experimental import pallas as pl
from jax.experimental.pallas import tpu as pltpu
from jax.experimental.pallas import tpu_sc as plsc

N_TOTAL = 100000
D_MODEL = 2048
D_EMB = 128
BATCH = 1024
M_NOISE = 4096
GAMMA = 0.5
TAU = 0.07
Z = np.float32(2876934.2 / 1281167 * N_TOTAL)
C0E = np.float32(M_NOISE * (1.0 / N_TOTAL) + 1e-07)
LOG_C0 = np.float32(np.log(M_NOISE * (1.0 / N_TOTAL)))

NC, NS = 2, 16
NW = NC * NS
NPAIR = BATCH * M_NOISE
NP_W = NPAIR // NW
CS = 8192
NCH = NP_W // CS
ROWS_W = BATCH // NW


def _tf2x32(k1, k2, x0, x1):
    k1 = np.uint32(k1)
    k2 = np.uint32(k2)
    ks = (k1, k2, np.uint32(k1 ^ k2 ^ np.uint32(0x1BD11BDA)))
    rot = (np.array([13, 15, 26, 6]), np.array([17, 29, 16, 24]))
    x0 = x0.astype(np.uint32) + ks[0]
    x1 = x1.astype(np.uint32) + ks[1]

    def rl(x, d):
        return (x << np.uint32(d)) | (x >> np.uint32(32 - d))

    for i in range(5):
        for r in rot[i % 2]:
            x0 = x0 + x1
            x1 = rl(x1, r)
            x1 = x0 ^ x1
        x0 = x0 + ks[(i + 1) % 3]
        x1 = x1 + ks[(i + 2) % 3] + np.uint32(i + 1)
    return x0, x1


def _np_randint_fixed_key(shape, n_total, seed=12345):
    size = int(np.prod(shape))
    b1, b2 = _tf2x32(np.uint32(seed >> 32), np.uint32(seed & 0xFFFFFFFF),
                     np.zeros(2, np.uint32), np.arange(2, dtype=np.uint32))
    subkeys = [(b1[i], b2[i]) for i in range(2)]

    def bits(key):
        a, b = _tf2x32(key[0], key[1], np.zeros(size, np.uint32),
                       np.arange(size, dtype=np.uint32))
        return a ^ b

    higher, lower = bits(subkeys[0]), bits(subkeys[1])
    span = np.uint32(n_total)
    with np.errstate(over="ignore"):
        m0 = np.uint32(2 ** 16) % span
        mult = (m0 * m0) % span
        off = ((higher % span) * mult + (lower % span)) % span
    return off.astype(np.int32).reshape(shape)


_RIDX = _np_randint_fixed_key((BATCH, M_NOISE), N_TOTAL)
_GIDX_FLAT = (
    _RIDX.astype(np.int64) * BATCH
    + np.arange(BATCH, dtype=np.int64)[:, None]
).astype(np.int32).reshape(-1)


def _emb_body(o_ref, w_ref, b_ref, emb_ref, embh_ref):
    x = lax.dot_general(o_ref[...], w_ref[...], (((1,), (1,)), ((), ())),
                        preferred_element_type=jnp.float32,
                        precision=lax.Precision.HIGHEST)
    x = x + b_ref[...]
    e = x / jnp.sqrt(jnp.sum(x * x, axis=1, keepdims=True))
    emb_ref[...] = e
    embh_ref[...] = e.astype(jnp.bfloat16)


def _emb_kernel(outputs, W, b2):
    return pl.pallas_call(
        _emb_body,
        out_shape=(jax.ShapeDtypeStruct((BATCH, D_EMB), jnp.float32),
                   jax.ShapeDtypeStruct((BATCH, D_EMB), jnp.bfloat16)),
    )(outputs, W, b2)


_KB = 2000


def _score_body(mb_ref, embh_ref, s_ref):
    s_ref[...] = lax.dot_general(
        mb_ref[...].astype(jnp.bfloat16), embh_ref[...],
        (((1,), (1,)), ((), ())), preferred_element_type=jnp.float32)


def _score_kernel(memory_bank, embh):
    return pl.pallas_call(
        _score_body,
        grid=(N_TOTAL // _KB,),
        in_specs=[pl.BlockSpec((_KB, D_EMB), lambda i: (i, 0)),
                  pl.BlockSpec((BATCH, D_EMB), lambda i: (0, 0))],
        out_specs=pl.BlockSpec((_KB, BATCH), lambda i: (i, 0)),
        out_shape=jax.ShapeDtypeStruct((N_TOTAL, BATCH), jnp.float32),
    )(memory_bank, embh)


@functools.lru_cache(maxsize=None)
def _noise_gather_kernel():
    mesh = plsc.VectorSubcoreMesh(core_axis_name="c", subcore_axis_name="s")

    @functools.partial(
        pl.kernel,
        mesh=mesh,
        out_type=jax.ShapeDtypeStruct((NPAIR,), jnp.float32),
        scratch_types=[
            pltpu.VMEM((CS,), jnp.int32),
            pltpu.VMEM((CS,), jnp.float32),
            pltpu.SemaphoreType.DMA,
        ],
    )
    def _noise_gather(s_hbm, gidx_hbm, out_hbm, idx_v, val_v, sem):
        wid = lax.axis_index("s") * NC + lax.axis_index("c")

        def chunk(ci, carry):
            base = wid * NP_W + ci * CS
            pltpu.sync_copy(gidx_hbm.at[pl.ds(base, CS)], idx_v)
            pltpu.async_copy(s_hbm.at[idx_v], val_v, sem).wait()
            pltpu.sync_copy(val_v, out_hbm.at[pl.ds(base, CS)])
            return carry

        lax.fori_loop(0, NCH, chunk, 0)

    return _noise_gather


@functools.lru_cache(maxsize=None)
def _row_gather_kernel():
    mesh = plsc.VectorSubcoreMesh(core_axis_name="c", subcore_axis_name="s")

    @functools.partial(
        pl.kernel,
        mesh=mesh,
        out_type=jax.ShapeDtypeStruct((BATCH, D_EMB), jnp.float32),
        scratch_types=[
            pltpu.VMEM((ROWS_W,), jnp.int32),
            pltpu.VMEM((ROWS_W, D_EMB), jnp.float32),
            pltpu.SemaphoreType.DMA,
        ],
    )
    def _row_gather(mb_hbm, idx_hbm, out_hbm, idx_v, rows_v, sem):
        wid = lax.axis_index("s") * NC + lax.axis_index("c")
        base = wid * ROWS_W
        pltpu.sync_copy(idx_hbm.at[pl.ds(base, ROWS_W)], idx_v)
        pltpu.async_copy(mb_hbm.at[idx_v], rows_v, sem).wait()
        pltpu.sync_copy(rows_v, out_hbm.at[pl.ds(base, ROWS_W)])

    return _row_gather


def _final_body(emb_ref, md_ref, ns_ref, entries_ref, sums_ref):
    e = emb_ref[...]
    md = md_ref[...]
    data_ip = jnp.sum(e * md, axis=1)
    dp = jnp.exp(data_ip / TAU) / Z
    sum_ld = jnp.sum(jnp.log(dp) - jnp.log(dp + C0E))
    ns = ns_ref[...]
    npr = jnp.exp(ns / TAU) / Z
    sum_ln = jnp.sum(LOG_C0 - jnp.log(npr + C0E))
    upd = GAMMA * md + (1.0 - GAMMA) * e
    entries_ref[...] = upd / jnp.sqrt(jnp.sum(upd * upd, axis=1,
                                              keepdims=True))
    row = lax.broadcasted_iota(jnp.int32, (8, 128), 0)
    sums_ref[...] = jnp.where(row == 0, sum_ld, sum_ln)


def _final_kernel(emb, mem_data, noise_s):
    return pl.pallas_call(
        _final_body,
        out_shape=(jax.ShapeDtypeStruct((BATCH, D_EMB), jnp.float32),
                   jax.ShapeDtypeStruct((8, 128), jnp.float32)),
    )(emb, mem_data, noise_s)


def kernel(outputs, indices, memory_bank, W, b):
    emb, embh = _emb_kernel(outputs, W, b.reshape(1, D_EMB))
    s = _score_kernel(memory_bank, embh)
    noise_flat = _noise_gather_kernel()(s.reshape(-1), jnp.asarray(_GIDX_FLAT))
    mem_data = _row_gather_kernel()(memory_bank, indices)
    entries, sums = _final_kernel(emb, mem_data,
                                  noise_flat.reshape(BATCH, M_NOISE))
    sum_ld = sums[0, 0]
    sum_ln = sums[1, 0]
    loss = jnp.reshape(-(sum_ld + sum_ln) / BATCH, (1,))
    data_loss = jnp.reshape(-sum_ld / BATCH, (1,))
    noise_loss = jnp.reshape(-sum_ln / BATCH, (1,))
    return (loss, entries, data_loss, noise_loss)


# speedup vs baseline: 43.5273x; 1.7157x over previous
import functools

import numpy as np
import jax
import jax.numpy as jnp
from jax import lax
from jax.experimental import pallas as pl
from jax.experimental.pallas import tpu as pltpu
from jax.experimental.pallas import tpu_sc as plsc

N_TOTAL = 100000
D_MODEL = 2048
D_EMB = 128
BATCH = 1024
M_NOISE = 4096
GAMMA = 0.5
TAU = 0.07
Z = np.float32(2876934.2 / 1281167 * N_TOTAL)
C0E = np.float32(M_NOISE * (1.0 / N_TOTAL) + 1e-07)
LOG_C0 = np.float32(np.log(M_NOISE * (1.0 / N_TOTAL)))

NC, NS = 2, 16
NW = NC * NS
NPAIR = BATCH * M_NOISE
NP_W = NPAIR // NW
CS = 8192
NCH = NP_W // CS
ROWS_W = BATCH // NW


def _tf2x32(k1, k2, x0, x1):
    k1 = np.uint32(k1)
    k2 = np.uint32(k2)
    ks = (k1, k2, np.uint32(k1 ^ k2 ^ np.uint32(0x1BD11BDA)))
    rot = (np.array([13, 15, 26, 6]), np.array([17, 29, 16, 24]))
    x0 = x0.astype(np.uint32) + ks[0]
    x1 = x1.astype(np.uint32) + ks[1]

    def rl(x, d):
        return (x << np.uint32(d)) | (x >> np.uint32(32 - d))

    for i in range(5):
        for r in rot[i % 2]:
            x0 = x0 + x1
            x1 = rl(x1, r)
            x1 = x0 ^ x1
        x0 = x0 + ks[(i + 1) % 3]
        x1 = x1 + ks[(i + 2) % 3] + np.uint32(i + 1)
    return x0, x1


def _np_randint_fixed_key(shape, n_total, seed=12345):
    size = int(np.prod(shape))
    b1, b2 = _tf2x32(np.uint32(seed >> 32), np.uint32(seed & 0xFFFFFFFF),
                     np.zeros(2, np.uint32), np.arange(2, dtype=np.uint32))
    subkeys = [(b1[i], b2[i]) for i in range(2)]

    def bits(key):
        a, b = _tf2x32(key[0], key[1], np.zeros(size, np.uint32),
                       np.arange(size, dtype=np.uint32))
        return a ^ b

    higher, lower = bits(subkeys[0]), bits(subkeys[1])
    span = np.uint32(n_total)
    with np.errstate(over="ignore"):
        m0 = np.uint32(2 ** 16) % span
        mult = (m0 * m0) % span
        off = ((higher % span) * mult + (lower % span)) % span
    return off.astype(np.int32).reshape(shape)


_RIDX = _np_randint_fixed_key((BATCH, M_NOISE), N_TOTAL)
_GIDX_FLAT = (
    _RIDX.astype(np.int64) * BATCH
    + np.arange(BATCH, dtype=np.int64)[:, None]
).astype(np.int32).reshape(-1)


def _emb_body(o_ref, w_ref, b_ref, emb_ref, embh_ref):
    x = lax.dot_general(o_ref[...], w_ref[...], (((1,), (1,)), ((), ())),
                        preferred_element_type=jnp.float32,
                        precision=lax.Precision.HIGHEST)
    x = x + b_ref[...]
    e = x / jnp.sqrt(jnp.sum(x * x, axis=1, keepdims=True))
    emb_ref[...] = e
    embh_ref[...] = e.astype(jnp.bfloat16)


def _emb_kernel(outputs, W, b2):
    return pl.pallas_call(
        _emb_body,
        out_shape=(jax.ShapeDtypeStruct((BATCH, D_EMB), jnp.float32),
                   jax.ShapeDtypeStruct((BATCH, D_EMB), jnp.bfloat16)),
    )(outputs, W, b2)


_KB = 2000


def _score_body(mb_ref, embh_ref, s_ref):
    s = lax.dot_general(
        mb_ref[...].astype(jnp.bfloat16), embh_ref[...],
        (((1,), (1,)), ((), ())), preferred_element_type=jnp.float32)
    s_ref[...] = s.reshape(_KB * BATCH)


def _score_kernel(memory_bank, embh):
    return pl.pallas_call(
        _score_body,
        grid=(N_TOTAL // _KB,),
        in_specs=[pl.BlockSpec((_KB, D_EMB), lambda i: (i, 0)),
                  pl.BlockSpec((BATCH, D_EMB), lambda i: (0, 0))],
        out_specs=pl.BlockSpec((_KB * BATCH,), lambda i: (i,)),
        out_shape=jax.ShapeDtypeStruct((N_TOTAL * BATCH,), jnp.float32),
    )(memory_bank, embh)


@functools.lru_cache(maxsize=None)
def _noise_gather_kernel():
    mesh = plsc.VectorSubcoreMesh(core_axis_name="c", subcore_axis_name="s")

    @functools.partial(
        pl.kernel,
        mesh=mesh,
        out_type=jax.ShapeDtypeStruct((NPAIR,), jnp.float32),
        scratch_types=[
            pltpu.VMEM((CS,), jnp.int32),
            pltpu.VMEM((CS,), jnp.float32),
            pltpu.SemaphoreType.DMA,
        ],
    )
    def _noise_gather(s_hbm, gidx_hbm, out_hbm, idx_v, val_v, sem):
        wid = lax.axis_index("s") * NC + lax.axis_index("c")

        def chunk(ci, carry):
            base = wid * NP_W + ci * CS
            pltpu.sync_copy(gidx_hbm.at[pl.ds(base, CS)], idx_v)
            pltpu.async_copy(s_hbm.at[idx_v], val_v, sem).wait()
            pltpu.sync_copy(val_v, out_hbm.at[pl.ds(base, CS)])
            return carry

        lax.fori_loop(0, NCH, chunk, 0)

    return _noise_gather


@functools.lru_cache(maxsize=None)
def _row_gather_kernel():
    mesh = plsc.VectorSubcoreMesh(core_axis_name="c", subcore_axis_name="s")

    @functools.partial(
        pl.kernel,
        mesh=mesh,
        out_type=jax.ShapeDtypeStruct((BATCH, D_EMB), jnp.float32),
        scratch_types=[
            pltpu.VMEM((ROWS_W,), jnp.int32),
            pltpu.VMEM((ROWS_W, D_EMB), jnp.float32),
            pltpu.SemaphoreType.DMA,
        ],
    )
    def _row_gather(mb_hbm, idx_hbm, out_hbm, idx_v, rows_v, sem):
        wid = lax.axis_index("s") * NC + lax.axis_index("c")
        base = wid * ROWS_W
        pltpu.sync_copy(idx_hbm.at[pl.ds(base, ROWS_W)], idx_v)
        pltpu.async_copy(mb_hbm.at[idx_v], rows_v, sem).wait()
        pltpu.sync_copy(rows_v, out_hbm.at[pl.ds(base, ROWS_W)])

    return _row_gather


def _final_body(emb_ref, md_ref, ns_ref, entries_ref, sums_ref):
    e = emb_ref[...]
    md = md_ref[...]
    data_ip = jnp.sum(e * md, axis=1)
    dp = jnp.exp(data_ip / TAU) / Z
    sum_ld = jnp.sum(jnp.log(dp) - jnp.log(dp + C0E))
    ns = ns_ref[...]
    npr = jnp.exp(ns / TAU) / Z
    sum_ln = jnp.sum(LOG_C0 - jnp.log(npr + C0E))
    upd = GAMMA * md + (1.0 - GAMMA) * e
    entries_ref[...] = upd / jnp.sqrt(jnp.sum(upd * upd, axis=1,
                                              keepdims=True))
    row = lax.broadcasted_iota(jnp.int32, (8, 128), 0)
    sums_ref[...] = jnp.where(row == 0, sum_ld, sum_ln)


def _final_kernel(emb, mem_data, noise_s):
    return pl.pallas_call(
        _final_body,
        out_shape=(jax.ShapeDtypeStruct((BATCH, D_EMB), jnp.float32),
                   jax.ShapeDtypeStruct((8, 128), jnp.float32)),
    )(emb, mem_data, noise_s)


def kernel(outputs, indices, memory_bank, W, b):
    emb, embh = _emb_kernel(outputs, W, b.reshape(1, D_EMB))
    s = _score_kernel(memory_bank, embh)
    noise_flat = _noise_gather_kernel()(s, jnp.asarray(_GIDX_FLAT))
    mem_data = _row_gather_kernel()(memory_bank, indices)
    entries, sums = _final_kernel(emb, mem_data,
                                  noise_flat.reshape(BATCH, M_NOISE))
    sum_ld = sums[0, 0]
    sum_ln = sums[1, 0]
    loss = jnp.reshape(-(sum_ld + sum_ln) / BATCH, (1,))
    data_loss = jnp.reshape(-sum_ld / BATCH, (1,))
    noise_loss = jnp.reshape(-sum_ln / BATCH, (1,))
    return (loss, entries, data_loss, noise_loss)
